# baseline (speedup 1.0000x reference)
import functools

import jax
import jax.numpy as jnp
from jax import lax
from jax.experimental import pallas as pl
from jax.experimental.pallas import tpu as pltpu
from jax.experimental.pallas import tpu_sc as plsc
from jax._src.pallas import mpmd as _mpmd

N, D, ATT, LV, NG, L, T = 50000, 256, 128, 4, 1024, 32, 4
NC, NS = 2, 16
NW = NC * NS


def _worker_id():
    return lax.axis_index("s") * NC + lax.axis_index("c")


def _make_gather(B, CH):
    b_per_w = B // NW
    n_ch = b_per_w // CH
    mesh = plsc.VectorSubcoreMesh(core_axis_name="c", subcore_axis_name="s")

    nb = min(3, n_ch)

    @functools.partial(
        pl.kernel,
        out_type=jax.ShapeDtypeStruct((B, D), jnp.float32),
        mesh=mesh,
        scratch_types=[
            pltpu.VMEM((b_per_w,), jnp.int32),
            pltpu.VMEM((nb, CH, D), jnp.float32),
            pltpu.SemaphoreType.DMA((nb,)),
            pltpu.SemaphoreType.DMA((nb,)),
        ],
    )
    def gk(table_hbm, idx_hbm, out_hbm, idx_v, rows_v, gsem, wsem):
        base = _worker_id() * b_per_w
        pltpu.sync_copy(idx_hbm.at[pl.ds(base, b_per_w)], idx_v)

        def g_desc(c):
            return pltpu.make_async_copy(
                table_hbm.at[idx_v.at[pl.ds(c * CH, CH)]],
                rows_v.at[c % nb], gsem.at[c % nb])

        def w_desc(c):
            return pltpu.make_async_copy(
                rows_v.at[c % nb], out_hbm.at[pl.ds(base + c * CH, CH)],
                wsem.at[c % nb])

        waited = set()
        for c in range(n_ch):
            if c >= nb:
                w_desc(c - nb).wait()
                waited.add(c - nb)
            g_desc(c).start()
            if c >= 1:
                g_desc(c - 1).wait()
                w_desc(c - 1).start()
        g_desc(n_ch - 1).wait()
        w_desc(n_ch - 1).start()
        for c in range(n_ch):
            if c not in waited:
                w_desc(c).wait()

    return gk


def _make_scatter():
    rows = NG // NW
    mesh = plsc.VectorSubcoreMesh(core_axis_name="c", subcore_axis_name="s")

    def sk(w_hbm, u_hbm, tgt_hbm, src_hbm, wout_hbm, tgt_v, src_v, rows_v, sem):
        base = _worker_id() * rows
        pltpu.sync_copy(tgt_hbm.at[pl.ds(base, rows)], tgt_v)
        pltpu.sync_copy(src_hbm.at[pl.ds(base, rows)], src_v)
        pltpu.async_copy(u_hbm.at[src_v], rows_v, sem).wait()
        pltpu.async_copy(rows_v, wout_hbm.at[tgt_v], sem).wait()

    return _mpmd._mpmd_map(
        [(mesh, sk)],
        jax.ShapeDtypeStruct((N, D), jnp.float32),
        input_output_aliases={0: 0},
        scratch_types=[
            pltpu.VMEM((rows,), jnp.int32),
            pltpu.VMEM((rows,), jnp.int32),
            pltpu.VMEM((rows, D), jnp.float32),
            pltpu.SemaphoreType.DMA,
        ],
    )


def _win_body(idxr_ref, idxc_ref, out_ref):
    jmat = lax.broadcasted_iota(jnp.int32, (NG, NG), 1)
    for i in range(2 * LV):
        row = idxr_ref[i : i + 1, :]
        col = idxc_ref[:, i : i + 1]
        cand = jnp.where(row == col, jmat, -1)
        out_ref[:, i : i + 1] = jnp.max(cand, axis=1, keepdims=True)


_win_call = pl.pallas_call(
    _win_body,
    out_shape=jax.ShapeDtypeStruct((NG, 2 * LV), jnp.int32),
)


GB = 128
GBL = GB * L
_NPB = NG * L // GBL


def _att_body(p_ref, c_ref, mask_ref, watt_ref, b_ref, v_ref, mt_ref, u_ref):
    p = p_ref[...]
    c = c_ref[...]
    wp = watt_ref[:D, :]
    wc = watt_ref[D:, :]
    t = jnp.dot(p, wp, preferred_element_type=jnp.float32)
    t = t + jnp.dot(c, wc, preferred_element_type=jnp.float32)
    t = t + b_ref[...]
    m = jnp.where(t >= 0.0, t, t * 0.01)
    pre = jnp.dot(m, v_ref[...], preferred_element_type=jnp.float32)
    pre3 = pre.reshape(GB, L, 1) + mask_ref[...][:, :, None]
    mx = jnp.max(pre3, axis=1, keepdims=True)
    e3 = jnp.exp(pre3 - mx)
    a3 = e3 / jnp.sum(e3, axis=1, keepdims=True)
    s_t = mt_ref[...] * a3.reshape(GBL, 1)
    temp = lax.dot_general(s_t, c, (((0,), (0,)), ((), ())),
                           preferred_element_type=jnp.float32)
    p0 = p.reshape(GB, L, D)[:, 0, :]
    u_ref[...] = (temp + p0) * 0.5


_att_call = pl.pallas_call(
    _att_body,
    out_shape=jax.ShapeDtypeStruct((NG, D), jnp.float32),
    grid=(NG // GB,),
    in_specs=[
        pl.BlockSpec((GBL, D), lambda i: (i, 0)),
        pl.BlockSpec((GBL, D), lambda i: (i + _NPB, 0)),
        pl.BlockSpec((GB, L), lambda i: (i, 0)),
        pl.BlockSpec((2 * D, ATT), lambda i: (0, 0)),
        pl.BlockSpec((1, ATT), lambda i: (0, 0)),
        pl.BlockSpec((ATT, 1), lambda i: (0, 0)),
        pl.BlockSpec((GBL, GB), lambda i: (0, 0)),
    ],
    out_specs=pl.BlockSpec((GB, D), lambda i: (i, 0)),
)


MB = 512


def _gru_body(x0_ref, x1_ref, x2_ref, x3_ref, c0_ref, wih_ref, whh_ref,
              bih_ref, bhh_ref, u_ref):
    xs = (x0_ref, x1_ref, x2_ref, x3_ref)
    wih = wih_ref[...]
    whh = whh_ref[...]
    bih = bih_ref[...]
    bhh = bhh_ref[...]
    h = jnp.zeros((MB, D), jnp.float32)
    for t in range(T):
        x = xs[t][...]
        gi = lax.dot_general(x, wih, (((1,), (1,)), ((), ())),
                             preferred_element_type=jnp.float32) + bih
        gh = lax.dot_general(h, whh, (((1,), (1,)), ((), ())),
                             preferred_element_type=jnp.float32) + bhh
        r = jax.nn.sigmoid(gi[:, :D] + gh[:, :D])
        z = jax.nn.sigmoid(gi[:, D:2 * D] + gh[:, D:2 * D])
        n = jnp.tanh(gi[:, 2 * D:] + r * gh[:, 2 * D:])
        h = (1.0 - z) * n + z * h
    u_ref[...] = (h + c0_ref[...]) * 0.5


_NMB = NG // MB

_gru_call = pl.pallas_call(
    _gru_body,
    out_shape=jax.ShapeDtypeStruct((NG, D), jnp.float32),
    grid=(_NMB,),
    in_specs=[
        pl.BlockSpec((MB, D), lambda i: (0 * _NMB + i, 0)),
        pl.BlockSpec((MB, D), lambda i: (1 * _NMB + i, 0)),
        pl.BlockSpec((MB, D), lambda i: (2 * _NMB + i, 0)),
        pl.BlockSpec((MB, D), lambda i: (3 * _NMB + i, 0)),
        pl.BlockSpec((MB, D), lambda i: (T * _NMB + i, 0)),
        pl.BlockSpec((3 * D, D), lambda i: (0, 0)),
        pl.BlockSpec((3 * D, D), lambda i: (0, 0)),
        pl.BlockSpec((1, 3 * D), lambda i: (0, 0)),
        pl.BlockSpec((1, 3 * D), lambda i: (0, 0)),
    ],
    out_specs=pl.BlockSpec((MB, D), lambda i: (i, 0)),
)


def kernel(W_emb, p2c_parent, p2c_children, p2c_mask, c2p_parents, c2p_child,
           c2p_mask, W_attention, b_attention, v_attention, gru_W_ih,
           gru_W_hh, gru_b_ih, gru_b_hh):
    W = W_emb
    gather_big = _make_gather(2 * NG * L, 128)
    gather_small = _make_gather((T + 1) * NG, 80)
    scatter = _make_scatter()

    tgt8 = jnp.concatenate(
        [p2c_parent[:, :, 0], c2p_child[:, :, 0]], axis=0).astype(jnp.int32)
    src8_t = _win_call(tgt8, tgt8.T)

    b1 = b_attention.reshape(1, ATT)
    v1 = v_attention.reshape(ATT, 1)
    bih = gru_b_ih.reshape(1, 3 * D)
    bhh = gru_b_hh.reshape(1, 3 * D)
    mt = (jnp.arange(GBL, dtype=jnp.int32)[:, None] // L
          == jnp.arange(GB, dtype=jnp.int32)[None, :]).astype(jnp.float32)

    Wg0 = lax.optimization_barrier(W_emb)

    for i in range(LV):
        idx = jnp.concatenate([p2c_parent[i].reshape(-1),
                               p2c_children[i].reshape(-1)]).astype(jnp.int32)
        PC = gather_big(Wg0 if i == 0 else W, idx)
        U = _att_call(PC, PC, p2c_mask[i], W_attention, b1, v1, mt)
        W = scatter(W, U, tgt8[i], src8_t[:, i])

    for i in range(LV):
        idx = jnp.concatenate([c2p_parents[i].T.reshape(-1),
                               c2p_child[i, :, 0]]).astype(jnp.int32)
        G = gather_small(W, idx)
        U = _gru_call(G, G, G, G, G, gru_W_ih, gru_W_hh, bih, bhh)
        W = scatter(W, U, tgt8[LV + i], src8_t[:, LV + i])

    return W


# speedup vs baseline: 1.0073x; 1.0073x over previous
import functools

import jax
import jax.numpy as jnp
from jax import lax
from jax.experimental import pallas as pl
from jax.experimental.pallas import tpu as pltpu
from jax.experimental.pallas import tpu_sc as plsc
from jax._src.pallas import mpmd as _mpmd

N, D, ATT, LV, NG, L, T = 50000, 256, 128, 4, 1024, 32, 4
NC, NS = 2, 16
NW = NC * NS


def _worker_id():
    return lax.axis_index("s") * NC + lax.axis_index("c")


def _make_gather(B, CH):
    b_per_w = B // NW
    n_ch = b_per_w // CH
    mesh = plsc.VectorSubcoreMesh(core_axis_name="c", subcore_axis_name="s")

    nb = min(3, n_ch)

    @functools.partial(
        pl.kernel,
        out_type=jax.ShapeDtypeStruct((B, D), jnp.float32),
        mesh=mesh,
        scratch_types=[
            pltpu.VMEM((b_per_w,), jnp.int32),
            pltpu.VMEM((nb, CH, D), jnp.float32),
            pltpu.SemaphoreType.DMA((nb,)),
            pltpu.SemaphoreType.DMA((nb,)),
        ],
    )
    def gk(table_hbm, idx_hbm, out_hbm, idx_v, rows_v, gsem, wsem):
        base = _worker_id() * b_per_w
        pltpu.sync_copy(idx_hbm.at[pl.ds(base, b_per_w)], idx_v)

        def g_desc(c):
            return pltpu.make_async_copy(
                table_hbm.at[idx_v.at[pl.ds(c * CH, CH)]],
                rows_v.at[c % nb], gsem.at[c % nb])

        def w_desc(c):
            return pltpu.make_async_copy(
                rows_v.at[c % nb], out_hbm.at[pl.ds(base + c * CH, CH)],
                wsem.at[c % nb])

        waited = set()
        for c in range(n_ch):
            if c >= nb:
                w_desc(c - nb).wait()
                waited.add(c - nb)
            g_desc(c).start()
            if c >= 1:
                g_desc(c - 1).wait()
                w_desc(c - 1).start()
        g_desc(n_ch - 1).wait()
        w_desc(n_ch - 1).start()
        for c in range(n_ch):
            if c not in waited:
                w_desc(c).wait()

    return gk


def _make_scatter():
    rows = NG // NW
    mesh = plsc.VectorSubcoreMesh(core_axis_name="c", subcore_axis_name="s")

    def sk(w_hbm, u_hbm, tgt_hbm, src_hbm, wout_hbm, tgt_v, src_v, rows_v, sem):
        base = _worker_id() * rows
        pltpu.sync_copy(tgt_hbm.at[pl.ds(base, rows)], tgt_v)
        pltpu.sync_copy(src_hbm.at[pl.ds(base, rows)], src_v)
        pltpu.async_copy(u_hbm.at[src_v], rows_v, sem).wait()
        pltpu.async_copy(rows_v, wout_hbm.at[tgt_v], sem).wait()

    return _mpmd._mpmd_map(
        [(mesh, sk)],
        jax.ShapeDtypeStruct((N, D), jnp.float32),
        input_output_aliases={0: 0},
        scratch_types=[
            pltpu.VMEM((rows,), jnp.int32),
            pltpu.VMEM((rows,), jnp.int32),
            pltpu.VMEM((rows, D), jnp.float32),
            pltpu.SemaphoreType.DMA,
        ],
    )


def _win_body(idxr_ref, idxc_ref, out_ref):
    jmat = lax.broadcasted_iota(jnp.int32, (NG, NG), 1)
    for i in range(2 * LV):
        row = idxr_ref[i : i + 1, :]
        col = idxc_ref[:, i : i + 1]
        cand = jnp.where(row == col, jmat, -1)
        out_ref[:, i : i + 1] = jnp.max(cand, axis=1, keepdims=True)


_win_call = pl.pallas_call(
    _win_body,
    out_shape=jax.ShapeDtypeStruct((NG, 2 * LV), jnp.int32),
)


GB = 128
GBL = GB * L
_NPB = NG * L // GBL


def _att_body(p_ref, c_ref, mask_ref, watt_ref, b_ref, v_ref, mt_ref, u_ref):
    p = p_ref[...]
    c = c_ref[...]
    wp = watt_ref[:D, :]
    wc = watt_ref[D:, :]
    t = jnp.dot(p, wp, preferred_element_type=jnp.float32)
    t = t + jnp.dot(c, wc, preferred_element_type=jnp.float32)
    t = t + b_ref[...]
    m = jnp.where(t >= 0.0, t, t * 0.01)
    pre = jnp.dot(m, v_ref[...], preferred_element_type=jnp.float32)
    pre3 = pre.reshape(GB, L, 1) + mask_ref[...][:, :, None]
    mx = jnp.max(pre3, axis=1, keepdims=True)
    e3 = jnp.exp(pre3 - mx)
    a3 = e3 / jnp.sum(e3, axis=1, keepdims=True)
    s_t = mt_ref[...] * a3.reshape(GBL, 1)
    temp = lax.dot_general(s_t, c, (((0,), (0,)), ((), ())),
                           preferred_element_type=jnp.float32)
    p0 = p.reshape(GB, L, D)[:, 0, :]
    u_ref[...] = (temp + p0) * 0.5


_att_call = pl.pallas_call(
    _att_body,
    out_shape=jax.ShapeDtypeStruct((NG, D), jnp.float32),
    grid=(NG // GB,),
    in_specs=[
        pl.BlockSpec((GBL, D), lambda i: (i, 0)),
        pl.BlockSpec((GBL, D), lambda i: (i + _NPB, 0)),
        pl.BlockSpec((GB, L), lambda i: (i, 0)),
        pl.BlockSpec((2 * D, ATT), lambda i: (0, 0)),
        pl.BlockSpec((1, ATT), lambda i: (0, 0)),
        pl.BlockSpec((ATT, 1), lambda i: (0, 0)),
        pl.BlockSpec((GBL, GB), lambda i: (0, 0)),
    ],
    out_specs=pl.BlockSpec((GB, D), lambda i: (i, 0)),
)


MB = 512


def _gru_body(x0_ref, x1_ref, x2_ref, x3_ref, c0_ref, wih_ref, whh_ref,
              bih_ref, bhh_ref, u_ref):
    xs = (x0_ref, x1_ref, x2_ref, x3_ref)
    wih = wih_ref[...]
    whh = whh_ref[...]
    bih = bih_ref[...]
    bhh = bhh_ref[...]
    h = jnp.zeros((MB, D), jnp.float32)
    for t in range(T):
        x = xs[t][...]
        gi = lax.dot_general(x, wih, (((1,), (1,)), ((), ())),
                             preferred_element_type=jnp.float32) + bih
        gh = lax.dot_general(h, whh, (((1,), (1,)), ((), ())),
                             preferred_element_type=jnp.float32) + bhh
        r = jax.nn.sigmoid(gi[:, :D] + gh[:, :D])
        z = jax.nn.sigmoid(gi[:, D:2 * D] + gh[:, D:2 * D])
        n = jnp.tanh(gi[:, 2 * D:] + r * gh[:, 2 * D:])
        h = (1.0 - z) * n + z * h
    u_ref[...] = (h + c0_ref[...]) * 0.5


_NMB = NG // MB

_gru_call = pl.pallas_call(
    _gru_body,
    out_shape=jax.ShapeDtypeStruct((NG, D), jnp.float32),
    grid=(_NMB,),
    in_specs=[
        pl.BlockSpec((MB, D), lambda i: (0 * _NMB + i, 0)),
        pl.BlockSpec((MB, D), lambda i: (1 * _NMB + i, 0)),
        pl.BlockSpec((MB, D), lambda i: (2 * _NMB + i, 0)),
        pl.BlockSpec((MB, D), lambda i: (3 * _NMB + i, 0)),
        pl.BlockSpec((MB, D), lambda i: (T * _NMB + i, 0)),
        pl.BlockSpec((3 * D, D), lambda i: (0, 0)),
        pl.BlockSpec((3 * D, D), lambda i: (0, 0)),
        pl.BlockSpec((1, 3 * D), lambda i: (0, 0)),
        pl.BlockSpec((1, 3 * D), lambda i: (0, 0)),
    ],
    out_specs=pl.BlockSpec((MB, D), lambda i: (i, 0)),
)


def kernel(W_emb, p2c_parent, p2c_children, p2c_mask, c2p_parents, c2p_child,
           c2p_mask, W_attention, b_attention, v_attention, gru_W_ih,
           gru_W_hh, gru_b_ih, gru_b_hh):
    W = W_emb
    gather_big = _make_gather(2 * NG * L, 128)
    gather_small = _make_gather((T + 1) * NG, 80)
    scatter = _make_scatter()

    tgt8 = jnp.concatenate(
        [p2c_parent[:, :, 0], c2p_child[:, :, 0]], axis=0).astype(jnp.int32)
    src8_t = _win_call(tgt8, tgt8.T)

    b1 = b_attention.reshape(1, ATT)
    v1 = v_attention.reshape(ATT, 1)
    bih = gru_b_ih.reshape(1, 3 * D)
    bhh = gru_b_hh.reshape(1, 3 * D)
    mt = (jnp.arange(GBL, dtype=jnp.int32)[:, None] // L
          == jnp.arange(GB, dtype=jnp.int32)[None, :]).astype(jnp.float32)

    for i in range(LV):
        idx = jnp.concatenate([p2c_parent[i].reshape(-1),
                               p2c_children[i].reshape(-1)]).astype(jnp.int32)
        PC = gather_big(W, idx)
        U = _att_call(PC, PC, p2c_mask[i], W_attention, b1, v1, mt)
        W = scatter(W, U, tgt8[i], src8_t[:, i])

    for i in range(LV):
        idx = jnp.concatenate([c2p_parents[i].T.reshape(-1),
                               c2p_child[i, :, 0]]).astype(jnp.int32)
        G = gather_small(W, idx)
        U = _gru_call(G, G, G, G, G, gru_W_ih, gru_W_hh, bih, bhh)
        W = scatter(W, U, tgt8[LV + i], src8_t[:, LV + i])

    return W
